# baseline (speedup 1.0000x reference)
import functools
import jax
import jax.numpy as jnp
from jax import lax
from jax.experimental import pallas as pl
from jax.experimental.pallas import tpu as pltpu
from jax.experimental.pallas import tpu_sc as plsc

N_BINS_K = 15
ROW_BLOCK = 8000
NSEG = 3000
HOFF = 3008
HSTRIDE = 6017
NW = 32
CHUNK = 2000


def _dense_body(nblocks, total_rows):
    def body(logits_ref, labels_ref, uppers_ref, conf_ref, seg_ref):
        x = logits_ref[...]
        b, c = x.shape
        xt = x.T
        m = jnp.max(xt, axis=0, keepdims=True)
        s = jnp.sum(jnp.exp(xt - m), axis=0, keepdims=True)
        conf = 1.0 / s
        cls_iota = jax.lax.broadcasted_iota(jnp.int32, (c, b), 0)
        cand = jnp.where(xt == m, cls_iota, c)
        pred = jnp.min(cand, axis=0, keepdims=True)
        labels = labels_ref[...][:, 0, :]
        acc_i = (pred == labels).astype(jnp.int32)

        uppers = uppers_ref[...][0]
        bin_idx = jnp.minimum(
            jnp.sum((uppers < conf).astype(jnp.int32), axis=0, keepdims=True),
            N_BINS_K - 1)
        seg2 = pred * N_BINS_K + bin_idx + (N_BINS_K * 100) * acc_i
        if nblocks * b != total_rows:
            i = pl.program_id(0)
            col = i * b + jax.lax.broadcasted_iota(jnp.int32, (1, b), 1)
            seg2 = jnp.where(col < total_rows, seg2, NSEG)
        conf_ref[...] = conf[None]
        seg_ref[...] = seg2[None]
    return body


def _sc_hist_body(nchunks):
    def body(seg_hbm, conf_hbm, out_hbm, seg_v, conf_v, hist_v, out_v):
        cid = lax.axis_index("c")
        sid = lax.axis_index("s")
        wid = sid * 2 + cid
        lane = lax.iota(jnp.int32, 16)
        lane_base = lane * HSTRIDE
        zv = jnp.zeros((16,), jnp.float32)
        ones = jnp.ones((16,), jnp.float32)

        def zero_body(i, _):
            hist_v[pl.ds(i * 16, 16)] = zv
            return 0
        lax.fori_loop(0, HSTRIDE, zero_body, 0)

        def chunk_body(t, _):
            k = t * NW + wid

            @pl.when(k < nchunks)
            def _():
                base = k * CHUNK
                pltpu.sync_copy(seg_hbm.at[pl.ds(base, CHUNK)], seg_v)
                pltpu.sync_copy(conf_hbm.at[pl.ds(base, CHUNK)], conf_v)

                def inner(j, _):
                    sv = seg_v[pl.ds(j * 16, 16)]
                    cv = conf_v[pl.ds(j * 16, 16)]
                    idx = sv + lane_base
                    plsc.addupdate_scatter(hist_v, [idx], ones)
                    plsc.addupdate_scatter(hist_v, [idx + HOFF], cv)
                    return 0
                lax.fori_loop(0, CHUNK // 16, inner, 0)
            return 0
        lax.fori_loop(0, (nchunks + NW - 1) // NW, chunk_body, 0)

        def merge_body(j, _):
            acc = hist_v[pl.ds(j * 16, 16)]
            for l in range(1, 16):
                acc = acc + hist_v[pl.ds(l * HSTRIDE + j * 16, 16)]
            out_v[pl.ds(j * 16, 16)] = acc
            return 0
        lax.fori_loop(0, 376, merge_body, 0)
        pltpu.sync_copy(out_v, out_hbm.at[wid])
    return body


def _final_body(parts_ref, out_ref):
    p = parts_ref[...]
    h = jnp.sum(p, axis=0, keepdims=True)
    l2 = N_BINS_K * 100
    cnt_lo = h[:, 0:l2]
    cnt_hi = h[:, l2:2 * l2]
    conf_lo = h[:, HOFF:HOFF + l2]
    conf_hi = h[:, HOFF + l2:HOFF + 2 * l2]
    count = cnt_lo + cnt_hi
    acc_sum = cnt_hi
    conf_sum = conf_lo + conf_hi
    jj = jax.lax.broadcasted_iota(jnp.int32, (l2, 100), 0)
    cc = jax.lax.broadcasted_iota(jnp.int32, (l2, 100), 1)
    sel = jnp.where(jj // N_BINS_K == cc, 1.0, 0.0)
    class_count = jax.lax.dot_general(
        count, sel, (((1,), (0,)), ((), ())),
        preferred_element_type=jnp.float32)
    ccnt_b = jax.lax.dot_general(
        jnp.maximum(class_count, 1.0), sel, (((1,), (1,)), ((), ())),
        preferred_element_type=jnp.float32)
    safe = jnp.maximum(count, 1.0)
    gap = jnp.where(count > 0.0,
                    jnp.abs(conf_sum / safe - acc_sum / safe) * count / ccnt_b,
                    0.0)
    out_ref[...] = jax.lax.dot_general(
        gap, sel, (((1,), (0,)), ((), ())),
        preferred_element_type=jnp.float32)


def kernel(logits, labels):
    n, c = logits.shape
    labels = labels.astype(jnp.int32)

    b = ROW_BLOCK
    nblocks = -(-n // b)
    npad = nblocks * b
    if npad != n:
        logits = jnp.pad(logits, ((0, npad - n), (0, 0)))
        labels = jnp.pad(labels, (0, npad - n))
    labels3 = labels.reshape(nblocks, 1, b)
    uppers = jnp.linspace(0.0, 1.0, N_BINS_K + 1)[1:].astype(
        jnp.float32).reshape(1, N_BINS_K, 1)

    conf3, seg3 = pl.pallas_call(
        _dense_body(nblocks, n),
        grid=(nblocks,),
        in_specs=[
            pl.BlockSpec((b, c), lambda i: (i, 0)),
            pl.BlockSpec((1, 1, b), lambda i: (i, 0, 0)),
            pl.BlockSpec((1, N_BINS_K, 1), lambda i: (0, 0, 0)),
        ],
        out_specs=[
            pl.BlockSpec((1, 1, b), lambda i: (i, 0, 0)),
            pl.BlockSpec((1, 1, b), lambda i: (i, 0, 0)),
        ],
        out_shape=[
            jax.ShapeDtypeStruct((nblocks, 1, b), jnp.float32),
            jax.ShapeDtypeStruct((nblocks, 1, b), jnp.int32),
        ],
    )(logits, labels3, uppers)
    conf_flat = conf3.reshape(npad)
    seg_flat = seg3.reshape(npad)

    nchunks = npad // CHUNK
    mesh = plsc.VectorSubcoreMesh(core_axis_name="c", subcore_axis_name="s")
    sc_hist = functools.partial(
        pl.kernel,
        mesh=mesh,
        compiler_params=pltpu.CompilerParams(needs_layout_passes=False),
        out_type=jax.ShapeDtypeStruct((NW, 6016), jnp.float32),
        scratch_types=[
            pltpu.VMEM((CHUNK,), jnp.int32),
            pltpu.VMEM((CHUNK,), jnp.float32),
            pltpu.VMEM((16 * HSTRIDE,), jnp.float32),
            pltpu.VMEM((6016,), jnp.float32),
        ],
    )(_sc_hist_body(nchunks))
    parts = sc_hist(seg_flat, conf_flat)

    out = pl.pallas_call(
        _final_body,
        out_shape=jax.ShapeDtypeStruct((1, c), jnp.float32),
    )(parts)
    return out.reshape(c)


# speedup vs baseline: 1.1970x; 1.1970x over previous
import jax
import jax.numpy as jnp
from jax.experimental import pallas as pl
from jax.experimental.pallas import tpu as pltpu

N_BINS_K = 15
ROW_BLOCK = 8000
NSPLIT = 4


def _ece_body(nblocks, total_rows):
    def body(*refs):
        logit_refs = refs[:NSPLIT]
        label_refs = refs[NSPLIT:2 * NSPLIT]
        uppers_ref = refs[2 * NSPLIT]
        out_ref = refs[2 * NSPLIT + 1]
        hist_ref = refs[2 * NSPLIT + 2]
        i = pl.program_id(0)

        @pl.when(i == 0)
        def _init():
            hist_ref[...] = jnp.zeros_like(hist_ref)

        uppers = uppers_ref[...][0]
        lowers = uppers_ref[...][1]

        for j in range(NSPLIT):
            x = logit_refs[j][...]
            b, c = x.shape
            xt = x.T
            m = jnp.max(xt, axis=0, keepdims=True)
            s = jnp.sum(jnp.exp(xt - m), axis=0, keepdims=True)
            conf = 1.0 / s
            cls_iota = jax.lax.broadcasted_iota(jnp.int32, (c, b), 0)
            cand = jnp.where(xt == m, cls_iota, c)
            pred = jnp.min(cand, axis=0, keepdims=True)
            labels = label_refs[j][...][:, 0, :]
            accv = (pred == labels).astype(jnp.float32)

            binoh = jnp.logical_and(lowers < conf, conf <= uppers)
            if nblocks * b * NSPLIT != total_rows:
                col = ((i * NSPLIT + j) * b
                       + jax.lax.broadcasted_iota(jnp.int32, (1, b), 1))
                binoh = jnp.logical_and(binoh, col < total_rows)
            conf_hi = conf.astype(jnp.bfloat16).astype(jnp.float32)
            conf_lo = conf - conf_hi
            valsT = jnp.concatenate(
                [jnp.where(binoh, 1.0, 0.0),
                 jnp.where(binoh, accv, 0.0),
                 jnp.where(binoh, conf_hi, 0.0),
                 jnp.where(binoh, conf_lo, 0.0)], axis=0
            ).astype(jnp.bfloat16)

            ponehot = jnp.where(cand == pred, 1.0, 0.0).astype(jnp.bfloat16)

            hist_ref[...] += jax.lax.dot_general(
                ponehot, valsT, (((1,), (1,)), ((), ())),
                preferred_element_type=jnp.float32)

        @pl.when(i == nblocks - 1)
        def _fin():
            h = hist_ref[...]
            count = h[:, :N_BINS_K]
            acc_sum = h[:, N_BINS_K:2 * N_BINS_K]
            conf_sum = (h[:, 2 * N_BINS_K:3 * N_BINS_K]
                        + h[:, 3 * N_BINS_K:4 * N_BINS_K])
            class_count = jnp.sum(count, axis=1, keepdims=True)
            safe = jnp.maximum(count, 1.0)
            prop = count / jnp.maximum(class_count, 1.0)
            gap = jnp.where(count > 0.0,
                            jnp.abs(conf_sum / safe - acc_sum / safe) * prop,
                            0.0)
            out_ref[...] = jnp.sum(gap, axis=1)[None, :]

    return body


def kernel(logits, labels):
    n, c = logits.shape
    labels = labels.astype(jnp.int32)

    bs = ROW_BLOCK // NSPLIT
    nblocks = -(-n // ROW_BLOCK)
    npad = nblocks * ROW_BLOCK
    if npad != n:
        logits = jnp.pad(logits, ((0, npad - n), (0, 0)))
        labels = jnp.pad(labels, (0, npad - n))
    labels3 = labels.reshape(nblocks * NSPLIT, 1, bs)
    boundaries = jnp.linspace(0.0, 1.0, N_BINS_K + 1).astype(jnp.float32)
    uppers = boundaries[1:]
    lowers = jnp.concatenate(
        [jnp.full((1,), -jnp.inf, jnp.float32), boundaries[1:N_BINS_K]])
    uppers = uppers.at[N_BINS_K - 1].set(jnp.inf)
    bnds = jnp.stack([uppers, lowers]).reshape(2, N_BINS_K, 1)

    logit_specs = [
        pl.BlockSpec((bs, c), lambda i, j=j: (NSPLIT * i + j, 0))
        for j in range(NSPLIT)
    ]
    label_specs = [
        pl.BlockSpec((1, 1, bs), lambda i, j=j: (NSPLIT * i + j, 0, 0))
        for j in range(NSPLIT)
    ]

    out = pl.pallas_call(
        _ece_body(nblocks, n),
        grid=(nblocks,),
        in_specs=logit_specs + label_specs + [
            pl.BlockSpec((2, N_BINS_K, 1), lambda i: (0, 0, 0)),
        ],
        out_specs=pl.BlockSpec((1, c), lambda i: (0, 0)),
        out_shape=jax.ShapeDtypeStruct((1, c), jnp.float32),
        scratch_shapes=[pltpu.VMEM((c, 4 * N_BINS_K), jnp.float32)],
    )(*([logits] * NSPLIT), *([labels3] * NSPLIT), bnds)
    return out.reshape(c)


# speedup vs baseline: 1.2701x; 1.0610x over previous
import jax
import jax.numpy as jnp
from jax.experimental import pallas as pl
from jax.experimental.pallas import tpu as pltpu

N_BINS_K = 15
ROW_BLOCK = 20000
NSPLIT = 2


def _ece_body(nblocks, total_rows):
    def body(*refs):
        logit_refs = refs[:NSPLIT]
        label_refs = refs[NSPLIT:2 * NSPLIT]
        uppers_ref = refs[2 * NSPLIT]
        out_ref = refs[2 * NSPLIT + 1]
        hist_ref = refs[2 * NSPLIT + 2]
        i = pl.program_id(0)

        @pl.when(i == 0)
        def _init():
            hist_ref[...] = jnp.zeros_like(hist_ref)

        uppers = uppers_ref[...][0]
        lowers = uppers_ref[...][1]

        for j in range(NSPLIT):
            x = logit_refs[j][...]
            b, c = x.shape
            xt = x.T
            m = jnp.max(xt, axis=0, keepdims=True)
            s = jnp.sum(jnp.exp(xt - m), axis=0, keepdims=True)
            conf = 1.0 / s
            cls_iota = jax.lax.broadcasted_iota(jnp.int32, (c, b), 0)
            cand = jnp.where(xt == m, cls_iota, c)
            pred = jnp.min(cand, axis=0, keepdims=True)
            labels = label_refs[j][...][:, 0, :]
            accv = (pred == labels).astype(jnp.float32)

            binoh = jnp.logical_and(lowers < conf, conf <= uppers)
            if nblocks * b * NSPLIT != total_rows:
                col = ((i * NSPLIT + j) * b
                       + jax.lax.broadcasted_iota(jnp.int32, (1, b), 1))
                binoh = jnp.logical_and(binoh, col < total_rows)
            conf_hi = conf.astype(jnp.bfloat16).astype(jnp.float32)
            conf_lo = conf - conf_hi
            valsT = jnp.concatenate(
                [jnp.where(binoh, 1.0, 0.0),
                 jnp.where(binoh, accv, 0.0),
                 jnp.where(binoh, conf_hi, 0.0),
                 jnp.where(binoh, conf_lo, 0.0)], axis=0
            ).astype(jnp.bfloat16)

            ponehot = jnp.where(cand == pred, 1.0, 0.0).astype(jnp.bfloat16)

            hist_ref[...] += jax.lax.dot_general(
                ponehot, valsT, (((1,), (1,)), ((), ())),
                preferred_element_type=jnp.float32)

        @pl.when(i == nblocks - 1)
        def _fin():
            h = hist_ref[...]
            count = h[:, :N_BINS_K]
            acc_sum = h[:, N_BINS_K:2 * N_BINS_K]
            conf_sum = (h[:, 2 * N_BINS_K:3 * N_BINS_K]
                        + h[:, 3 * N_BINS_K:4 * N_BINS_K])
            class_count = jnp.sum(count, axis=1, keepdims=True)
            safe = jnp.maximum(count, 1.0)
            prop = count / jnp.maximum(class_count, 1.0)
            gap = jnp.where(count > 0.0,
                            jnp.abs(conf_sum / safe - acc_sum / safe) * prop,
                            0.0)
            out_ref[...] = jnp.sum(gap, axis=1)[None, :]

    return body


def kernel(logits, labels):
    n, c = logits.shape
    labels = labels.astype(jnp.int32)

    bs = ROW_BLOCK // NSPLIT
    nblocks = -(-n // ROW_BLOCK)
    npad = nblocks * ROW_BLOCK
    if npad != n:
        logits = jnp.pad(logits, ((0, npad - n), (0, 0)))
        labels = jnp.pad(labels, (0, npad - n))
    labels3 = labels.reshape(nblocks * NSPLIT, 1, bs)
    boundaries = jnp.linspace(0.0, 1.0, N_BINS_K + 1).astype(jnp.float32)
    uppers = boundaries[1:]
    lowers = jnp.concatenate(
        [jnp.full((1,), -jnp.inf, jnp.float32), boundaries[1:N_BINS_K]])
    uppers = uppers.at[N_BINS_K - 1].set(jnp.inf)
    bnds = jnp.stack([uppers, lowers]).reshape(2, N_BINS_K, 1)

    logit_specs = [
        pl.BlockSpec((bs, c), lambda i, j=j: (NSPLIT * i + j, 0))
        for j in range(NSPLIT)
    ]
    label_specs = [
        pl.BlockSpec((1, 1, bs), lambda i, j=j: (NSPLIT * i + j, 0, 0))
        for j in range(NSPLIT)
    ]

    out = pl.pallas_call(
        _ece_body(nblocks, n),
        grid=(nblocks,),
        in_specs=logit_specs + label_specs + [
            pl.BlockSpec((2, N_BINS_K, 1), lambda i: (0, 0, 0)),
        ],
        out_specs=pl.BlockSpec((1, c), lambda i: (0, 0)),
        out_shape=jax.ShapeDtypeStruct((1, c), jnp.float32),
        scratch_shapes=[pltpu.VMEM((c, 4 * N_BINS_K), jnp.float32)],
    )(*([logits] * NSPLIT), *([labels3] * NSPLIT), bnds)
    return out.reshape(c)


# speedup vs baseline: 1.2902x; 1.0158x over previous
import jax
import jax.numpy as jnp
from jax.experimental import pallas as pl
from jax.experimental.pallas import tpu as pltpu

N_BINS_K = 15
ROW_BLOCK = 40000
NSPLIT = 2


def _ece_body(nblocks, total_rows):
    def body(*refs):
        logit_refs = refs[:NSPLIT]
        label_refs = refs[NSPLIT:2 * NSPLIT]
        uppers_ref = refs[2 * NSPLIT]
        out_ref = refs[2 * NSPLIT + 1]
        hist_ref = refs[2 * NSPLIT + 2]
        i = pl.program_id(0)

        @pl.when(i == 0)
        def _init():
            hist_ref[...] = jnp.zeros_like(hist_ref)

        uppers = uppers_ref[...][0]
        lowers = uppers_ref[...][1]

        for j in range(NSPLIT):
            x = logit_refs[j][...]
            b, c = x.shape
            xt = x.T
            m = jnp.max(xt, axis=0, keepdims=True)
            s = jnp.sum(jnp.exp(xt - m), axis=0, keepdims=True)
            conf = 1.0 / s
            cls_iota = jax.lax.broadcasted_iota(jnp.int32, (c, b), 0)
            cand = jnp.where(xt == m, cls_iota, c)
            pred = jnp.min(cand, axis=0, keepdims=True)
            labels = label_refs[j][...][:, 0, :]
            accv = (pred == labels).astype(jnp.float32)

            binoh = jnp.logical_and(lowers < conf, conf <= uppers)
            if nblocks * b * NSPLIT != total_rows:
                col = ((i * NSPLIT + j) * b
                       + jax.lax.broadcasted_iota(jnp.int32, (1, b), 1))
                binoh = jnp.logical_and(binoh, col < total_rows)
            conf_hi = conf.astype(jnp.bfloat16).astype(jnp.float32)
            conf_lo = conf - conf_hi
            valsT = jnp.concatenate(
                [jnp.where(binoh, 1.0, 0.0),
                 jnp.where(binoh, accv, 0.0),
                 jnp.where(binoh, conf_hi, 0.0),
                 jnp.where(binoh, conf_lo, 0.0)], axis=0
            ).astype(jnp.bfloat16)

            ponehot = jnp.where(cand == pred, 1.0, 0.0).astype(jnp.bfloat16)

            hist_ref[...] += jax.lax.dot_general(
                ponehot, valsT, (((1,), (1,)), ((), ())),
                preferred_element_type=jnp.float32)

        @pl.when(i == nblocks - 1)
        def _fin():
            h = hist_ref[...]
            count = h[:, :N_BINS_K]
            acc_sum = h[:, N_BINS_K:2 * N_BINS_K]
            conf_sum = (h[:, 2 * N_BINS_K:3 * N_BINS_K]
                        + h[:, 3 * N_BINS_K:4 * N_BINS_K])
            class_count = jnp.sum(count, axis=1, keepdims=True)
            safe = jnp.maximum(count, 1.0)
            prop = count / jnp.maximum(class_count, 1.0)
            gap = jnp.where(count > 0.0,
                            jnp.abs(conf_sum / safe - acc_sum / safe) * prop,
                            0.0)
            out_ref[...] = jnp.sum(gap, axis=1)[None, :]

    return body


def kernel(logits, labels):
    n, c = logits.shape
    labels = labels.astype(jnp.int32)

    bs = ROW_BLOCK // NSPLIT
    nblocks = -(-n // ROW_BLOCK)
    npad = nblocks * ROW_BLOCK
    if npad != n:
        logits = jnp.pad(logits, ((0, npad - n), (0, 0)))
        labels = jnp.pad(labels, (0, npad - n))
    labels3 = labels.reshape(nblocks * NSPLIT, 1, bs)
    boundaries = jnp.linspace(0.0, 1.0, N_BINS_K + 1).astype(jnp.float32)
    uppers = boundaries[1:]
    lowers = jnp.concatenate(
        [jnp.full((1,), -jnp.inf, jnp.float32), boundaries[1:N_BINS_K]])
    uppers = uppers.at[N_BINS_K - 1].set(jnp.inf)
    bnds = jnp.stack([uppers, lowers]).reshape(2, N_BINS_K, 1)

    logit_specs = [
        pl.BlockSpec((bs, c), lambda i, j=j: (NSPLIT * i + j, 0))
        for j in range(NSPLIT)
    ]
    label_specs = [
        pl.BlockSpec((1, 1, bs), lambda i, j=j: (NSPLIT * i + j, 0, 0))
        for j in range(NSPLIT)
    ]

    out = pl.pallas_call(
        _ece_body(nblocks, n),
        grid=(nblocks,),
        in_specs=logit_specs + label_specs + [
            pl.BlockSpec((2, N_BINS_K, 1), lambda i: (0, 0, 0)),
        ],
        out_specs=pl.BlockSpec((1, c), lambda i: (0, 0)),
        out_shape=jax.ShapeDtypeStruct((1, c), jnp.float32),
        scratch_shapes=[pltpu.VMEM((c, 4 * N_BINS_K), jnp.float32)],
    )(*([logits] * NSPLIT), *([labels3] * NSPLIT), bnds)
    return out.reshape(c)


# speedup vs baseline: 1.2912x; 1.0007x over previous
import jax
import jax.numpy as jnp
from jax.experimental import pallas as pl
from jax.experimental.pallas import tpu as pltpu

N_BINS_K = 15
ROW_BLOCK = 50000
NSPLIT = 2


def _ece_body(nblocks, total_rows):
    def body(*refs):
        logit_refs = refs[:NSPLIT]
        label_refs = refs[NSPLIT:2 * NSPLIT]
        uppers_ref = refs[2 * NSPLIT]
        out_ref = refs[2 * NSPLIT + 1]
        hist_ref = refs[2 * NSPLIT + 2]
        i = pl.program_id(0)

        @pl.when(i == 0)
        def _init():
            hist_ref[...] = jnp.zeros_like(hist_ref)

        uppers = uppers_ref[...][0]
        lowers = uppers_ref[...][1]

        for j in range(NSPLIT):
            x = logit_refs[j][...]
            b, c = x.shape
            xt = x.T
            m = jnp.max(xt, axis=0, keepdims=True)
            s = jnp.sum(jnp.exp(xt - m), axis=0, keepdims=True)
            conf = 1.0 / s
            cls_iota = jax.lax.broadcasted_iota(jnp.int32, (c, b), 0)
            cand = jnp.where(xt == m, cls_iota, c)
            pred = jnp.min(cand, axis=0, keepdims=True)
            labels = label_refs[j][...][:, 0, :]
            accv = (pred == labels).astype(jnp.float32)

            binoh = jnp.logical_and(lowers < conf, conf <= uppers)
            if nblocks * b * NSPLIT != total_rows:
                col = ((i * NSPLIT + j) * b
                       + jax.lax.broadcasted_iota(jnp.int32, (1, b), 1))
                binoh = jnp.logical_and(binoh, col < total_rows)
            conf_hi = conf.astype(jnp.bfloat16).astype(jnp.float32)
            conf_lo = conf - conf_hi
            valsT = jnp.concatenate(
                [jnp.where(binoh, 1.0, 0.0),
                 jnp.where(binoh, accv, 0.0),
                 jnp.where(binoh, conf_hi, 0.0),
                 jnp.where(binoh, conf_lo, 0.0)], axis=0
            ).astype(jnp.bfloat16)

            ponehot = jnp.where(cand == pred, 1.0, 0.0).astype(jnp.bfloat16)

            hist_ref[...] += jax.lax.dot_general(
                ponehot, valsT, (((1,), (1,)), ((), ())),
                preferred_element_type=jnp.float32)

        @pl.when(i == nblocks - 1)
        def _fin():
            h = hist_ref[...]
            count = h[:, :N_BINS_K]
            acc_sum = h[:, N_BINS_K:2 * N_BINS_K]
            conf_sum = (h[:, 2 * N_BINS_K:3 * N_BINS_K]
                        + h[:, 3 * N_BINS_K:4 * N_BINS_K])
            class_count = jnp.sum(count, axis=1, keepdims=True)
            safe = jnp.maximum(count, 1.0)
            prop = count / jnp.maximum(class_count, 1.0)
            gap = jnp.where(count > 0.0,
                            jnp.abs(conf_sum / safe - acc_sum / safe) * prop,
                            0.0)
            out_ref[...] = jnp.sum(gap, axis=1)[None, :]

    return body


def kernel(logits, labels):
    n, c = logits.shape
    labels = labels.astype(jnp.int32)

    bs = ROW_BLOCK // NSPLIT
    nblocks = -(-n // ROW_BLOCK)
    npad = nblocks * ROW_BLOCK
    if npad != n:
        logits = jnp.pad(logits, ((0, npad - n), (0, 0)))
        labels = jnp.pad(labels, (0, npad - n))
    labels3 = labels.reshape(nblocks * NSPLIT, 1, bs)
    boundaries = jnp.linspace(0.0, 1.0, N_BINS_K + 1).astype(jnp.float32)
    uppers = boundaries[1:]
    lowers = jnp.concatenate(
        [jnp.full((1,), -jnp.inf, jnp.float32), boundaries[1:N_BINS_K]])
    uppers = uppers.at[N_BINS_K - 1].set(jnp.inf)
    bnds = jnp.stack([uppers, lowers]).reshape(2, N_BINS_K, 1)

    logit_specs = [
        pl.BlockSpec((bs, c), lambda i, j=j: (NSPLIT * i + j, 0))
        for j in range(NSPLIT)
    ]
    label_specs = [
        pl.BlockSpec((1, 1, bs), lambda i, j=j: (NSPLIT * i + j, 0, 0))
        for j in range(NSPLIT)
    ]

    out = pl.pallas_call(
        _ece_body(nblocks, n),
        grid=(nblocks,),
        in_specs=logit_specs + label_specs + [
            pl.BlockSpec((2, N_BINS_K, 1), lambda i: (0, 0, 0)),
        ],
        out_specs=pl.BlockSpec((1, c), lambda i: (0, 0)),
        out_shape=jax.ShapeDtypeStruct((1, c), jnp.float32),
        scratch_shapes=[pltpu.VMEM((c, 4 * N_BINS_K), jnp.float32)],
    )(*([logits] * NSPLIT), *([labels3] * NSPLIT), bnds)
    return out.reshape(c)
